# baseline (speedup 1.0000x reference)
import functools

import jax
import jax.numpy as jnp
from jax import lax
from jax.experimental import pallas as pl
from jax.experimental.pallas import tpu as pltpu
from jax.experimental.pallas import tpu_sc as plsc

V = 1000000
D = 64
L = 16
NC = 2
NS = 16
NW = NC * NS
NB = 4096
NT = 200
NBUF = 4
T = NT // NBUF
SCALE = 8.0
TBLK = 12800

_mesh = plsc.VectorSubcoreMesh(core_axis_name="c", subcore_axis_name="s")


def _detile_body(x_ref, o_ref):
    y = x_ref[...].T * SCALE
    o_ref[...] = jnp.concatenate([y, y], axis=1)


_detile = pl.pallas_call(
    _detile_body,
    grid=(pl.cdiv(V, TBLK),),
    in_specs=[pl.BlockSpec((D, TBLK), lambda i: (0, i))],
    out_specs=pl.BlockSpec((TBLK, 128), lambda i: (i, 0)),
    out_shape=jax.ShapeDtypeStruct((V, 128), jnp.float32),
)


@functools.partial(
    pl.kernel,
    mesh=_mesh,
    out_type=jax.ShapeDtypeStruct((NT, D // 8, NW, 8, 128), jnp.float32),
    scratch_types=[
        pltpu.VMEM((NT, 128), jnp.int32),
        pltpu.VMEM((NBUF, 128, 128), jnp.float32),
        pltpu.VMEM((NBUF, D // 8, 8, 128), jnp.float32),
        pltpu.SemaphoreType.DMA((NBUF,)),
        pltpu.SemaphoreType.DMA((NBUF,)),
    ],
    compiler_params=pltpu.CompilerParams(
        use_tc_tiling_on_sc=False, needs_layout_passes=False),
)
def _emb(table_hbm, tok_hbm, out_hbm, idx_v, rin, rob, sem_g, sem_w):
    wid = lax.axis_index("s") * NC + lax.axis_index("c")
    pltpu.sync_copy(tok_hbm.at[:, wid], idx_v)

    iota = lax.iota(jnp.int32, L)
    rows = [iota + bg * L for bg in range(128 // L)]

    def start_gather(t, b):
        pltpu.make_async_copy(
            table_hbm.at[idx_v.at[t]], rin.at[b], sem_g.at[b]).start()

    def wait_gather(b):
        pltpu.make_async_copy(
            table_hbm.at[pl.ds(0, 128)], rin.at[b], sem_g.at[b]).wait()

    def start_wb(t, b):
        pltpu.make_async_copy(
            rob.at[b], out_hbm.at[t, :, wid], sem_w.at[b]).start()

    def wait_wb(b):
        pltpu.make_async_copy(
            rob.at[b], out_hbm.at[0, :, wid], sem_w.at[b]).wait()

    def transpose_scale(b):
        @plsc.parallel_loop(0, D, unroll=2)
        def col(d):
            dcol = (iota + d) & (D - 1)
            dh = dcol >> 3
            dl = dcol & 7
            for bg in range(128 // L):
                vec = plsc.load_gather(rin.at[b], [rows[bg], dcol])
                plsc.store_scatter(rob.at[b], [dh, dl, rows[bg]], vec)

    for b in range(NBUF):
        start_gather(b, b)

    for b in range(NBUF):
        wait_gather(b)
        transpose_scale(b)
        start_wb(b, b)
        start_gather(b + NBUF, b)

    def mid(s, c):
        for b in range(NBUF):
            t = s * NBUF + b
            wait_gather(b)
            wait_wb(b)
            transpose_scale(b)
            start_wb(t, b)
            start_gather(t + NBUF, b)
        return c

    lax.fori_loop(1, T - 1, mid, 0)

    for b in range(NBUF):
        t = (T - 1) * NBUF + b
        wait_gather(b)
        wait_wb(b)
        transpose_scale(b)
        start_wb(t, b)

    for b in range(NBUF):
        wait_wb(b)


def kernel(tokens, table):
    tokT = tokens.T.reshape(NT, NW, 128).astype(jnp.int32)
    tab = _detile(table.T)
    out5 = _emb(tab, tokT)
    return out5.transpose(2, 4, 0, 1, 3).reshape(NB, NT, D)


# speedup vs baseline: 1.0385x; 1.0385x over previous
import functools

import jax
import jax.numpy as jnp
from jax import lax
from jax.experimental import pallas as pl
from jax.experimental.pallas import tpu as pltpu
from jax.experimental.pallas import tpu_sc as plsc

V = 1000000
D = 64
L = 16
NC = 2
NS = 16
NW = NC * NS
NB = 4096
NT = 200
NBUF = 4
T = NT // NBUF
SCALE = 8.0
TBLK = 25600

_mesh = plsc.VectorSubcoreMesh(core_axis_name="c", subcore_axis_name="s")


def _detile_body(x_ref, o_ref):
    y = x_ref[...].T * SCALE
    o_ref[...] = jnp.concatenate([y, y], axis=1)


_detile = pl.pallas_call(
    _detile_body,
    grid=(pl.cdiv(V, TBLK),),
    in_specs=[pl.BlockSpec((D, TBLK), lambda i: (0, i))],
    out_specs=pl.BlockSpec((TBLK, 128), lambda i: (i, 0)),
    out_shape=jax.ShapeDtypeStruct((V, 128), jnp.float32),
)


@functools.partial(
    pl.kernel,
    mesh=_mesh,
    out_type=jax.ShapeDtypeStruct((NT, D // 8, NW, 8, 128), jnp.float32),
    scratch_types=[
        pltpu.VMEM((NT, 128), jnp.int32),
        pltpu.VMEM((NBUF, 128, 128), jnp.float32),
        pltpu.VMEM((NBUF, D // 8, 8, 128), jnp.float32),
        pltpu.SemaphoreType.DMA((NBUF,)),
        pltpu.SemaphoreType.DMA((NBUF,)),
    ],
    compiler_params=pltpu.CompilerParams(
        use_tc_tiling_on_sc=False, needs_layout_passes=False),
)
def _emb(table_hbm, tok_hbm, out_hbm, idx_v, rin, rob, sem_g, sem_w):
    wid = lax.axis_index("s") * NC + lax.axis_index("c")
    pltpu.sync_copy(tok_hbm.at[:, wid], idx_v)

    iota = lax.iota(jnp.int32, L)
    rows = [iota + bg * L for bg in range(128 // L)]

    def start_gather(t, b):
        pltpu.make_async_copy(
            table_hbm.at[idx_v.at[t]], rin.at[b], sem_g.at[b]).start()

    def wait_gather(b):
        pltpu.make_async_copy(
            table_hbm.at[pl.ds(0, 128)], rin.at[b], sem_g.at[b]).wait()

    def start_wb(t, b):
        pltpu.make_async_copy(
            rob.at[b], out_hbm.at[t, :, wid], sem_w.at[b]).start()

    def wait_wb(b):
        pltpu.make_async_copy(
            rob.at[b], out_hbm.at[0, :, wid], sem_w.at[b]).wait()

    def transpose_scale(b):
        @plsc.parallel_loop(0, D, unroll=2)
        def col(d):
            dcol = (iota + d) & (D - 1)
            dh = dcol >> 3
            dl = dcol & 7
            for bg in range(128 // L):
                vec = plsc.load_gather(rin.at[b], [rows[bg], dcol])
                plsc.store_scatter(rob.at[b], [dh, dl, rows[bg]], vec)

    for b in range(NBUF):
        start_gather(b, b)

    for b in range(NBUF):
        wait_gather(b)
        transpose_scale(b)
        start_wb(b, b)
        start_gather(b + NBUF, b)

    def mid(s, c):
        for b in range(NBUF):
            t = s * NBUF + b
            wait_gather(b)
            wait_wb(b)
            transpose_scale(b)
            start_wb(t, b)
            start_gather(t + NBUF, b)
        return c

    lax.fori_loop(1, T - 1, mid, 0)

    for b in range(NBUF):
        t = (T - 1) * NBUF + b
        wait_gather(b)
        wait_wb(b)
        transpose_scale(b)
        start_wb(t, b)

    for b in range(NBUF):
        wait_wb(b)


def kernel(tokens, table):
    tokT = tokens.T.reshape(NT, NW, 128).astype(jnp.int32)
    tab = _detile(table.T)
    out5 = _emb(tab, tokT)
    return out5.transpose(2, 4, 0, 1, 3).reshape(NB, NT, D)
